# Initial kernel scaffold; baseline (speedup 1.0000x reference)
import functools

import jax
import jax.numpy as jnp
from jax import lax
from jax.experimental import pallas as pl
from jax.experimental.pallas import tpu as pltpu
from jax.experimental.pallas import tpu_sc as plsc

_DIM = 32
_NC, _NS = 2, 16
_NW = _NC * _NS
_GRP = 128
_G_PER_CHUNK = 10
_CHUNK = _GRP * _G_PER_CHUNK


def _make_gather(n_idx):
    assert n_idx % (_NW * _CHUNK) == 0
    n_chunks = n_idx // (_NW * _CHUNK)
    assert n_chunks % 2 == 0 and n_chunks >= 4
    g_per_w = n_chunks * _G_PER_CHUNK
    mesh = plsc.VectorSubcoreMesh(core_axis_name="c", subcore_axis_name="s")

    @functools.partial(
        pl.kernel,
        mesh=mesh,
        out_type=jax.ShapeDtypeStruct((n_idx, _DIM), jnp.float32),
        scratch_types=[
            pltpu.VMEM((g_per_w, _GRP), jnp.int32),
            pltpu.VMEM((_CHUNK, _DIM), jnp.float32),
            pltpu.VMEM((_CHUNK, _DIM), jnp.float32),
            pltpu.SemaphoreType.DMA,
            pltpu.SemaphoreType.DMA,
        ],
    )
    def gather_kernel(idx_hbm, table_hbm, out_hbm, idx_v, rows0, rows1,
                      sem0, sem1):
        wid = lax.axis_index("s") * _NC + lax.axis_index("c")
        out_base = wid * (n_chunks * _CHUNK)
        pltpu.sync_copy(idx_hbm.at[pl.ds(wid * g_per_w, g_per_w)], idx_v)

        bufs = (rows0, rows1)
        sems = (sem0, sem1)

        def _copies(c, slot):
            for j in range(_G_PER_CHUNK):
                yield pltpu.make_async_copy(
                    table_hbm.at[idx_v.at[c * _G_PER_CHUNK + j]],
                    bufs[slot].at[pl.ds(j * _GRP, _GRP)],
                    sems[slot],
                )

        def issue(c, slot):
            for cp in _copies(c, slot):
                cp.start()

        def drain(c, slot):
            for cp in _copies(c, slot):
                cp.wait()

        def flush(c, slot):
            pltpu.sync_copy(
                bufs[slot],
                out_hbm.at[pl.ds(out_base + c * _CHUNK, _CHUNK)])

        issue(0, 0)
        issue(1, 1)
        drain(0, 0)
        flush(0, 0)

        def body(t, carry):
            c = 2 * t + 1
            issue(c + 1, 0)
            drain(c, 1)
            flush(c, 1)
            issue(c + 2, 1)
            drain(c + 1, 0)
            flush(c + 1, 0)
            return carry

        lax.fori_loop(0, n_chunks // 2 - 1, body, 0)
        drain(n_chunks - 1, 1)
        flush(n_chunks - 1, 1)

    return gather_kernel


def kernel(data, ivectors):
    b, h = data.shape
    vocab, dim = ivectors.shape
    assert dim == _DIM
    n = b * h
    idx2 = data.reshape(n // _GRP, _GRP).astype(jnp.int32)
    out = _make_gather(n)(idx2, ivectors)
    return out.reshape(b, h, dim)


# speedup vs baseline: 1.1121x; 1.1121x over previous
import functools

import jax
import jax.numpy as jnp
from jax import lax
from jax.experimental import pallas as pl
from jax.experimental.pallas import tpu as pltpu
from jax.experimental.pallas import tpu_sc as plsc

_DIM = 32
_NC, _NS = 2, 16
_NW = _NC * _NS
_GRP = 128
_G_PER_CHUNK = 10
_CHUNK = _GRP * _G_PER_CHUNK


def _make_gather(n_idx):
    assert n_idx % (_NW * _CHUNK) == 0
    n_chunks = n_idx // (_NW * _CHUNK)
    assert n_chunks % 2 == 0 and n_chunks >= 4
    g_per_w = n_chunks * _G_PER_CHUNK
    mesh = plsc.VectorSubcoreMesh(core_axis_name="c", subcore_axis_name="s")

    @functools.partial(
        pl.kernel,
        mesh=mesh,
        out_type=jax.ShapeDtypeStruct((n_idx, _DIM), jnp.float32),
        compiler_params=pltpu.CompilerParams(use_tc_tiling_on_sc=False),
        scratch_types=[
            pltpu.VMEM((g_per_w, _GRP), jnp.int32),
            pltpu.VMEM((_CHUNK, _DIM), jnp.float32),
            pltpu.VMEM((_CHUNK, _DIM), jnp.float32),
            pltpu.SemaphoreType.DMA,
            pltpu.SemaphoreType.DMA,
        ],
    )
    def gather_kernel(idx_hbm, table_hbm, out_hbm, idx_v, rows0, rows1,
                      sem0, sem1):
        wid = lax.axis_index("s") * _NC + lax.axis_index("c")
        out_base = wid * (n_chunks * _CHUNK)
        pltpu.sync_copy(idx_hbm.at[pl.ds(wid * g_per_w, g_per_w)], idx_v)

        bufs = (rows0, rows1)
        sems = (sem0, sem1)

        def _copies(c, slot):
            for j in range(_G_PER_CHUNK):
                yield pltpu.make_async_copy(
                    table_hbm.at[idx_v.at[c * _G_PER_CHUNK + j]],
                    bufs[slot].at[pl.ds(j * _GRP, _GRP)],
                    sems[slot],
                )

        def issue(c, slot):
            for cp in _copies(c, slot):
                cp.start()

        def drain(c, slot):
            for cp in _copies(c, slot):
                cp.wait()

        def flush(c, slot):
            pltpu.sync_copy(
                bufs[slot],
                out_hbm.at[pl.ds(out_base + c * _CHUNK, _CHUNK)])

        issue(0, 0)
        issue(1, 1)
        drain(0, 0)
        flush(0, 0)

        def body(t, carry):
            c = 2 * t + 1
            issue(c + 1, 0)
            drain(c, 1)
            flush(c, 1)
            issue(c + 2, 1)
            drain(c + 1, 0)
            flush(c + 1, 0)
            return carry

        lax.fori_loop(0, n_chunks // 2 - 1, body, 0)
        drain(n_chunks - 1, 1)
        flush(n_chunks - 1, 1)

    return gather_kernel


def kernel(data, ivectors):
    b, h = data.shape
    vocab, dim = ivectors.shape
    assert dim == _DIM
    n = b * h
    idx2 = data.reshape(n // _GRP, _GRP).astype(jnp.int32)
    out = _make_gather(n)(idx2, ivectors)
    return out.reshape(b, h, dim)


# speedup vs baseline: 1.1127x; 1.0005x over previous
import functools

import jax
import jax.numpy as jnp
from jax import lax
from jax.experimental import pallas as pl
from jax.experimental.pallas import tpu as pltpu
from jax.experimental.pallas import tpu_sc as plsc

_DIM = 32
_NC, _NS = 2, 16
_NW = _NC * _NS
_GRP = 128
_G_PER_CHUNK = 10
_CHUNK = _GRP * _G_PER_CHUNK


def _make_gather(n_idx, vocab):
    assert n_idx % (_NW * _CHUNK) == 0
    n_chunks = n_idx // (_NW * _CHUNK)
    assert n_chunks % 2 == 0 and n_chunks >= 4
    n_per_w = n_chunks * _CHUNK
    mesh = plsc.VectorSubcoreMesh(core_axis_name="c", subcore_axis_name="s")

    @functools.partial(
        pl.kernel,
        mesh=mesh,
        out_type=jax.ShapeDtypeStruct((n_idx, _DIM), jnp.float32),
        compiler_params=pltpu.CompilerParams(use_tc_tiling_on_sc=False),
        scratch_types=[
            pltpu.VMEM((n_per_w,), jnp.int32),
            pltpu.VMEM((_CHUNK, _DIM), jnp.float32),
            pltpu.VMEM((_CHUNK, _DIM), jnp.float32),
            pltpu.SemaphoreType.DMA,
            pltpu.SemaphoreType.DMA,
        ],
    )
    def gather_kernel(idx_hbm, table_hbm, out_hbm, idx_v, rows0, rows1,
                      sem0, sem1):
        wid = lax.axis_index("s") * _NC + lax.axis_index("c")
        out_base = wid * n_per_w
        table2 = table_hbm
        pltpu.sync_copy(idx_hbm.at[pl.ds(wid * n_per_w, n_per_w)], idx_v)

        bufs = (rows0, rows1)
        sems = (sem0, sem1)

        def _copies(c, slot):
            for j in range(_G_PER_CHUNK):
                yield pltpu.make_async_copy(
                    table2.at[idx_v.at[pl.ds((c * _G_PER_CHUNK + j) * _GRP,
                                             _GRP)]],
                    bufs[slot].at[pl.ds(j * _GRP, _GRP)],
                    sems[slot],
                )

        def issue(c, slot):
            for cp in _copies(c, slot):
                cp.start()

        def drain(c, slot):
            for cp in _copies(c, slot):
                cp.wait()

        def flush(c, slot):
            pltpu.sync_copy(
                bufs[slot],
                out_hbm.at[pl.ds(out_base + c * _CHUNK, _CHUNK)])

        issue(0, 0)
        issue(1, 1)
        drain(0, 0)
        flush(0, 0)

        def body(t, carry):
            c = 2 * t + 1
            issue(c + 1, 0)
            drain(c, 1)
            flush(c, 1)
            issue(c + 2, 1)
            drain(c + 1, 0)
            flush(c + 1, 0)
            return carry

        lax.fori_loop(0, n_chunks // 2 - 1, body, 0)
        drain(n_chunks - 1, 1)
        flush(n_chunks - 1, 1)

    return gather_kernel


def kernel(data, ivectors):
    b, h = data.shape
    vocab, dim = ivectors.shape
    assert dim == _DIM
    n = b * h
    idx1 = data.reshape(n).astype(jnp.int32)
    out = _make_gather(n, vocab)(idx1, ivectors)
    return out.reshape(b, h, dim)


# speedup vs baseline: 1.7973x; 1.6153x over previous
import functools

import jax
import jax.numpy as jnp
from jax import lax
from jax.experimental import pallas as pl
from jax.experimental.pallas import tpu as pltpu
from jax.experimental.pallas import tpu_sc as plsc

_DIM = 32
_NC, _NS = 2, 16
_NW = _NC * _NS
_BCHUNK = 8


def _make_gather(batch, hist, vocab):
    assert batch % (_NW * _BCHUNK) == 0
    n_chunks = batch // (_NW * _BCHUNK)
    assert n_chunks % 2 == 0 and n_chunks >= 4
    b_per_w = n_chunks * _BCHUNK
    mesh = plsc.VectorSubcoreMesh(core_axis_name="c", subcore_axis_name="s")

    @functools.partial(
        pl.kernel,
        mesh=mesh,
        out_type=jax.ShapeDtypeStruct((batch, hist, _DIM), jnp.float32),
        compiler_params=pltpu.CompilerParams(use_tc_tiling_on_sc=False),
        scratch_types=[
            pltpu.VMEM((b_per_w, hist), jnp.int32),
            pltpu.VMEM((_BCHUNK, hist, _DIM), jnp.float32),
            pltpu.VMEM((_BCHUNK, hist, _DIM), jnp.float32),
            pltpu.SemaphoreType.DMA,
            pltpu.SemaphoreType.DMA,
        ],
    )
    def gather_kernel(idx_hbm, table_hbm, out_hbm, idx_v, rows0, rows1,
                      sem0, sem1):
        wid = lax.axis_index("s") * _NC + lax.axis_index("c")
        b_base = wid * b_per_w
        pltpu.sync_copy(idx_hbm.at[pl.ds(b_base, b_per_w)], idx_v)

        bufs = (rows0, rows1)
        sems = (sem0, sem1)

        def _copies(c, slot):
            for i in range(_BCHUNK):
                yield pltpu.make_async_copy(
                    table_hbm.at[idx_v.at[c * _BCHUNK + i]],
                    bufs[slot].at[i],
                    sems[slot],
                )

        def issue(c, slot):
            for cp in _copies(c, slot):
                cp.start()

        def drain(c, slot):
            for cp in _copies(c, slot):
                cp.wait()

        def flush(c, slot):
            pltpu.sync_copy(
                bufs[slot],
                out_hbm.at[pl.ds(b_base + c * _BCHUNK, _BCHUNK)])

        issue(0, 0)
        issue(1, 1)
        drain(0, 0)
        flush(0, 0)

        def body(t, carry):
            c = 2 * t + 1
            issue(c + 1, 0)
            drain(c, 1)
            flush(c, 1)
            issue(c + 2, 1)
            drain(c + 1, 0)
            flush(c + 1, 0)
            return carry

        lax.fori_loop(0, n_chunks // 2 - 1, body, 0)
        drain(n_chunks - 1, 1)
        flush(n_chunks - 1, 1)

    return gather_kernel


def kernel(data, ivectors):
    b, h = data.shape
    vocab, dim = ivectors.shape
    assert dim == _DIM
    idx = data.astype(jnp.int32)
    return _make_gather(b, h, vocab)(idx, ivectors)
